# baseline (speedup 1.0000x reference)
import functools

import jax
import jax.numpy as jnp
import numpy as np
from jax import lax
from jax.experimental import pallas as pl
from jax.experimental.pallas import tpu as pltpu
from jax.experimental.pallas import tpu_sc as plsc

H = 8
DH = 16
L = 16
NC = 2
NS = 16
NW = NC * NS
ACCW = 136


def _pack_rows(x):
    u = lax.bitcast_convert_type(x, jnp.uint32)
    rne = (u + 0x7FFF + ((u >> 16) & 1)) >> 16
    lo = rne[:, :64]
    hi = rne[:, 64:]
    return lax.bitcast_convert_type(lo | (hi << 16), jnp.int32)


def _qkv_body(x_ref, wq_ref, wk_ref, wv_ref, q_ref, k_ref, v_ref):
    xb = x_ref[...]
    scale = 1.0 / np.sqrt(DH)
    q_ref[...] = _pack_rows(
        jnp.dot(xb, wq_ref[...], preferred_element_type=jnp.float32) * scale)
    k_ref[...] = _pack_rows(
        jnp.dot(xb, wk_ref[...], preferred_element_type=jnp.float32))
    v_ref[...] = _pack_rows(
        jnp.dot(xb, wv_ref[...], preferred_element_type=jnp.float32))


def _qkv_call(x, WQ, WK, WV, bn):
    n, d = x.shape
    grid = (n // bn,)
    node_spec = pl.BlockSpec((bn, d), lambda i: (i, 0))
    packed_spec = pl.BlockSpec((bn, d // 2), lambda i: (i, 0))
    w_spec = pl.BlockSpec((d, d), lambda i: (0, 0))
    out = jax.ShapeDtypeStruct((n, d // 2), jnp.int32)
    return pl.pallas_call(
        _qkv_body,
        grid=grid,
        in_specs=[node_spec, w_spec, w_spec, w_spec],
        out_specs=[packed_spec, packed_spec, packed_spec],
        out_shape=[out, out, out],
    )(x, WQ, WK, WV)


def _proj_body(ea_ref, we_ref, p_ref):
    ea = ea_ref[...].astype(jnp.bfloat16)
    we = we_ref[...].astype(jnp.bfloat16)
    p_ref[...] = _pack_rows(
        jnp.dot(ea, we, preferred_element_type=jnp.float32))


def _proj_call(edge_attr, WE, be):
    e, d = edge_attr.shape
    return pl.pallas_call(
        _proj_body,
        grid=(e // be,),
        in_specs=[pl.BlockSpec((be, d), lambda i: (i, 0)),
                  pl.BlockSpec((d, d), lambda i: (0, 0))],
        out_specs=pl.BlockSpec((be, d // 2), lambda i: (i, 0)),
        out_shape=jax.ShapeDtypeStruct((e, d // 2), jnp.int32),
    )(edge_attr, WE)


def _sc_edge_call(q, k, v, p, ei, n_nodes, d, ch):
    e_edges = p.shape[0]
    dp = d // 2
    epw = e_edges // NW
    nchunk = epw // ch
    npair = nchunk // 2
    rows_pt = n_nodes // NS

    mesh = plsc.VectorSubcoreMesh(core_axis_name="c", subcore_axis_name="s",
                                  num_cores=NC, num_subcores=NS)

    @functools.partial(
        pl.kernel,
        out_type=(jax.ShapeDtypeStruct((e_edges, d), jnp.float32),
                  jax.ShapeDtypeStruct((NC, n_nodes, ACCW), jnp.float32)),
        mesh=mesh,
        scratch_types=[
            pltpu.VMEM((2, 2, ch), jnp.int32),
            pltpu.VMEM((2, ch, dp), jnp.int32),
            pltpu.VMEM((2, ch, dp), jnp.int32),
            pltpu.VMEM((2, ch, dp), jnp.int32),
            pltpu.VMEM((2, ch, dp), jnp.int32),
            pltpu.VMEM((ch, d), jnp.float32),
            pltpu.VMEM((2, ch, ACCW), jnp.float32),
            pltpu.VMEM_SHARED((n_nodes, ACCW), jnp.float32),
            pltpu.SemaphoreType.DMA,
            pltpu.SemaphoreType.DMA,
            pltpu.SemaphoreType.DMA,
            pltpu.SemaphoreType.DMA,
            pltpu.SemaphoreType.DMA,
            pltpu.SemaphoreType.DMA,
            pltpu.SemaphoreType.DMA,
            pltpu.SemaphoreType.DMA,
            pltpu.SemaphoreType.DMA,
            pltpu.SemaphoreType.DMA,
            pltpu.SemaphoreType.DMA,
        ],
        compiler_params=pltpu.CompilerParams(use_tc_tiling_on_sc=False,
                                             needs_layout_passes=False),
    )
    def sc_kernel(q_hbm, k_hbm, v_hbm, p_hbm, ei_hbm,
                  e_hbm, acc_hbm,
                  idx, kg, qg, pg, vg, pb, ab, acc,
                  semk0, semk1, semq0, semq1, semp0, semp1,
                  semv0, semv1, seme, sems0, sems1):
        semk = (semk0, semk1)
        semq = (semq0, semq1)
        semp = (semp0, semp1)
        semv = (semv0, semv1)
        sems = (sems0, sems1)
        cid = lax.axis_index("c")
        sid = lax.axis_index("s")
        wid = sid * NC + cid
        zero = jnp.zeros((L,), jnp.float32)
        lane0 = lax.iota(jnp.int32, L) == 0
        tail_cols = d + jnp.minimum(lax.iota(jnp.int32, L), 7)
        tail_msk = lax.iota(jnp.int32, L) < 8
        himask = jnp.int32(-65536)

        @plsc.parallel_loop(0, ch)
        def zrow(i):
            for j in range(d // L):
                ab[0, i, pl.ds(j * L, L)] = zero
            plsc.store_scatter(ab.at[0],
                               [jnp.full((L,), i, jnp.int32), tail_cols],
                               zero, mask=tail_msk)

        nzc = rows_pt // ch
        zrem = rows_pt - nzc * ch

        def zacc(r, _):
            pltpu.sync_copy(ab.at[0],
                            acc.at[pl.ds(sid * rows_pt + r * ch, ch)])
            return 0
        lax.fori_loop(0, nzc, zacc, 0, unroll=False)
        if zrem:
            pltpu.sync_copy(
                ab.at[0, pl.ds(0, zrem)],
                acc.at[pl.ds(sid * rows_pt + nzc * ch, zrem)])
        plsc.subcore_barrier()

        base0 = wid * epw

        def issue_chunk(t, slot):
            base = base0 + t * ch
            pltpu.sync_copy(ei_hbm.at[:, pl.ds(base, ch)], idx.at[slot])
            pltpu.async_copy(k_hbm.at[idx.at[slot, 0]], kg.at[slot],
                             semk[slot])
            pltpu.async_copy(q_hbm.at[idx.at[slot, 1]], qg.at[slot],
                             semq[slot])
            pltpu.async_copy(v_hbm.at[idx.at[slot, 0]], vg.at[slot],
                             semv[slot])
            pltpu.async_copy(p_hbm.at[pl.ds(base, ch)], pg.at[slot],
                             semp[slot])

        issue_chunk(0, 0)

        def unpack(w):
            return (plsc.bitcast(w << 16, jnp.float32),
                    plsc.bitcast(w & himask, jnp.float32))

        def body(pair, b):
            t = pair * 2 + b
            bn = b ^ 1
            base = base0 + t * ch

            def wait_prev_s():
                pltpu.make_async_copy(
                    ab.at[bn], acc.at[idx.at[bn, 1]], sems[bn]).wait()
            if b == 1:
                wait_prev_s()
            else:
                pl.when(pair >= 1)(wait_prev_s)

            def prefetch():
                issue_chunk(t + 1, bn)
            if b == 0:
                prefetch()
            else:
                pl.when(pair < npair - 1)(prefetch)

            pltpu.make_async_copy(k_hbm.at[idx.at[b, 0]], kg.at[b],
                                  semk[b]).wait()
            pltpu.make_async_copy(q_hbm.at[idx.at[b, 1]], qg.at[b],
                                  semq[b]).wait()
            pltpu.make_async_copy(v_hbm.at[idx.at[b, 0]], vg.at[b],
                                  semv[b]).wait()
            pltpu.make_async_copy(p_hbm.at[pl.ds(base, ch)], pg.at[b],
                                  semp[b]).wait()

            def wait_prev_e():
                pltpu.make_async_copy(
                    pb, e_hbm.at[pl.ds(base - ch, ch)], seme).wait()
            if b == 1:
                wait_prev_e()
            else:
                pl.when(pair >= 1)(wait_prev_e)

            @plsc.parallel_loop(0, ch, unroll=4)
            def p1(e):
                row = jnp.full((L,), e, jnp.int32)
                for j in range(H // 2):
                    sl = pl.ds(j * L, L)
                    klo, khi = unpack(kg[b, e, sl])
                    qlo, qhi = unpack(qg[b, e, sl])
                    plo, phi = unpack(pg[b, e, sl])
                    vlo, vhi = unpack(vg[b, e, sl])
                    sc_lo = klo * qlo * plo
                    sc_hi = khi * qhi * phi
                    pb[e, sl] = sc_lo
                    pb[e, pl.ds((j + 4) * L, L)] = sc_hi
                    r_lo = jnp.sum(sc_lo)
                    r_hi = jnp.sum(sc_hi)
                    s_lo = jnp.exp(jnp.full(
                        (L,), jnp.minimum(jnp.maximum(r_lo, -5.0), 5.0),
                        jnp.float32))
                    s_hi = jnp.exp(jnp.full(
                        (L,), jnp.minimum(jnp.maximum(r_hi, -5.0), 5.0),
                        jnp.float32))
                    ab[b, e, sl] = vlo * s_lo
                    ab[b, e, pl.ds((j + 4) * L, L)] = vhi * s_hi
                    plsc.store_scatter(
                        ab.at[b], [row, jnp.full((L,), d + j, jnp.int32)],
                        s_lo, mask=lane0)
                    plsc.store_scatter(
                        ab.at[b],
                        [row, jnp.full((L,), d + j + 4, jnp.int32)],
                        s_hi, mask=lane0)

            pltpu.async_copy(pb, e_hbm.at[pl.ds(base, ch)], seme)
            pltpu.async_copy(ab.at[b], acc.at[idx.at[b, 1]], sems[b],
                             add=True)

        def pair_loop(pair, _):
            body(pair, 0)
            body(pair, 1)
            return 0
        lax.fori_loop(0, npair, pair_loop, 0, unroll=False)

        pltpu.make_async_copy(
            pb, e_hbm.at[pl.ds(base0 + (nchunk - 1) * ch, ch)], seme).wait()
        pltpu.make_async_copy(ab.at[1], acc.at[idx.at[1, 1]], sems[1]).wait()

        plsc.subcore_barrier()

        def outcp(r, _):
            sl = pl.ds(sid * rows_pt + r * ch, ch)
            pltpu.sync_copy(acc.at[sl], ab.at[0])
            pltpu.sync_copy(ab.at[0], acc_hbm.at[cid, sl])
            return 0
        lax.fori_loop(0, nzc, outcp, 0, unroll=False)
        if zrem:
            sl = pl.ds(sid * rows_pt + nzc * ch, zrem)
            pltpu.sync_copy(acc.at[sl], ab.at[0, pl.ds(0, zrem)])
            pltpu.sync_copy(ab.at[0, pl.ds(0, zrem)], acc_hbm.at[cid, sl])

    return sc_kernel(q, k, v, p, ei)


def _final_body(acc_ref, s_ref, h_ref):
    a = acc_ref[0] + acc_ref[1]
    wv = a[:, 0:128]
    z8 = a[:, 128:136]
    zfull = jnp.dot(z8, s_ref[...], preferred_element_type=jnp.float32)
    h_ref[...] = wv / (zfull + 1e-6)


def _final_call(acc, bn):
    _, n, _ = acc.shape
    sel = np.zeros((H, 128), np.float32)
    for h in range(H):
        sel[h, h * DH:(h + 1) * DH] = 1.0
    sel = jnp.asarray(sel)
    return pl.pallas_call(
        _final_body,
        grid=(n // bn,),
        in_specs=[pl.BlockSpec((NC, bn, ACCW), lambda i: (0, i, 0)),
                  pl.BlockSpec((H, 128), lambda i: (0, 0))],
        out_specs=pl.BlockSpec((bn, 128), lambda i: (i, 0)),
        out_shape=jax.ShapeDtypeStruct((n, 128), jnp.float32),
    )(acc, sel)


def kernel(x, edge_attr, edge_index, WQ, WK, WV, WE):
    n, d = x.shape

    q, k, v = _qkv_call(x, WQ, WK, WV, bn=2000)
    p = _proj_call(edge_attr, WE, be=8000)
    e_out, acc = _sc_edge_call(q, k, v, p, edge_index, n, d, ch=40)
    h = _final_call(acc, bn=1000)
    return (h, e_out)


# speedup vs baseline: 2.0572x; 2.0572x over previous
import functools

import jax
import jax.numpy as jnp
import numpy as np
from jax import lax
from jax.experimental import pallas as pl
from jax.experimental.pallas import tpu as pltpu
from jax.experimental.pallas import tpu_sc as plsc

H = 8
DH = 16
L = 16
NC = 2
NS = 16
NW = NC * NS
ACCW = 136


def _pack_rows(x):
    u = lax.bitcast_convert_type(x, jnp.uint32)
    rne = (u + 0x7FFF + ((u >> 16) & 1)) >> 16
    lo = rne[:, :64]
    hi = rne[:, 64:]
    return lax.bitcast_convert_type(lo | (hi << 16), jnp.int32)


def _qkv_body(x_ref, wq_ref, wk_ref, wv_ref, q_ref, k_ref, v_ref):
    xb = x_ref[...]
    scale = 1.0 / np.sqrt(DH)
    q_ref[...] = _pack_rows(
        jnp.dot(xb, wq_ref[...], preferred_element_type=jnp.float32) * scale)
    k_ref[...] = _pack_rows(
        jnp.dot(xb, wk_ref[...], preferred_element_type=jnp.float32))
    v_ref[...] = _pack_rows(
        jnp.dot(xb, wv_ref[...], preferred_element_type=jnp.float32))


def _qkv_call(x, WQ, WK, WV, bn):
    n, d = x.shape
    grid = (n // bn,)
    node_spec = pl.BlockSpec((bn, d), lambda i: (i, 0))
    packed_spec = pl.BlockSpec((bn, d // 2), lambda i: (i, 0))
    w_spec = pl.BlockSpec((d, d), lambda i: (0, 0))
    out = jax.ShapeDtypeStruct((n, d // 2), jnp.int32)
    return pl.pallas_call(
        _qkv_body,
        grid=grid,
        in_specs=[node_spec, w_spec, w_spec, w_spec],
        out_specs=[packed_spec, packed_spec, packed_spec],
        out_shape=[out, out, out],
    )(x, WQ, WK, WV)


def _proj_body(ea_ref, we_ref, p_ref):
    ea = ea_ref[...].astype(jnp.bfloat16)
    we = we_ref[...].astype(jnp.bfloat16)
    p_ref[...] = _pack_rows(
        jnp.dot(ea, we, preferred_element_type=jnp.float32))


def _proj_call(edge_attr, WE, be):
    e, d = edge_attr.shape
    return pl.pallas_call(
        _proj_body,
        grid=(e // be,),
        in_specs=[pl.BlockSpec((be, d), lambda i: (i, 0)),
                  pl.BlockSpec((d, d), lambda i: (0, 0))],
        out_specs=pl.BlockSpec((be, d // 2), lambda i: (i, 0)),
        out_shape=jax.ShapeDtypeStruct((e, d // 2), jnp.int32),
    )(edge_attr, WE)


def _sc_edge_call(q, k, v, p, ei, n_nodes, d, ch):
    e_edges = p.shape[0]
    dp = d // 2
    epw = e_edges // NW
    nchunk = epw // ch
    npair = nchunk // 2
    rows_pt = n_nodes // NS

    mesh = plsc.VectorSubcoreMesh(core_axis_name="c", subcore_axis_name="s",
                                  num_cores=NC, num_subcores=NS)

    @functools.partial(
        pl.kernel,
        out_type=(jax.ShapeDtypeStruct((e_edges, d), jnp.float32),
                  jax.ShapeDtypeStruct((NC, n_nodes, ACCW), jnp.float32)),
        mesh=mesh,
        scratch_types=[
            pltpu.VMEM((2, 2, ch), jnp.int32),
            pltpu.VMEM((2, ch, dp), jnp.int32),
            pltpu.VMEM((2, ch, dp), jnp.int32),
            pltpu.VMEM((2, ch, dp), jnp.int32),
            pltpu.VMEM((2, ch, dp), jnp.int32),
            pltpu.VMEM((ch, d), jnp.float32),
            pltpu.VMEM((2, ch, ACCW), jnp.float32),
            pltpu.VMEM_SHARED((n_nodes, ACCW), jnp.float32),
            pltpu.SemaphoreType.DMA,
            pltpu.SemaphoreType.DMA,
            pltpu.SemaphoreType.DMA,
            pltpu.SemaphoreType.DMA,
            pltpu.SemaphoreType.DMA,
            pltpu.SemaphoreType.DMA,
            pltpu.SemaphoreType.DMA,
            pltpu.SemaphoreType.DMA,
            pltpu.SemaphoreType.DMA,
            pltpu.SemaphoreType.DMA,
            pltpu.SemaphoreType.DMA,
        ],
        compiler_params=pltpu.CompilerParams(use_tc_tiling_on_sc=False,
                                             needs_layout_passes=False),
    )
    def sc_kernel(q_hbm, k_hbm, v_hbm, p_hbm, ei_hbm,
                  e_hbm, acc_hbm,
                  idx, kg, qg, pg, vg, pb, ab, acc,
                  semk0, semk1, semq0, semq1, semp0, semp1,
                  semv0, semv1, seme, sems0, sems1):
        semk = (semk0, semk1)
        semq = (semq0, semq1)
        semp = (semp0, semp1)
        semv = (semv0, semv1)
        sems = (sems0, sems1)
        cid = lax.axis_index("c")
        sid = lax.axis_index("s")
        wid = sid * NC + cid
        zero = jnp.zeros((L,), jnp.float32)
        lane0 = lax.iota(jnp.int32, L) == 0
        tail_cols = d + jnp.minimum(lax.iota(jnp.int32, L), 7)
        tail_msk = lax.iota(jnp.int32, L) < 8
        himask = jnp.int32(-65536)

        @plsc.parallel_loop(0, ch)
        def zrow(i):
            for j in range(d // L):
                ab[0, i, pl.ds(j * L, L)] = zero
            plsc.store_scatter(ab.at[0],
                               [jnp.full((L,), i, jnp.int32), tail_cols],
                               zero, mask=tail_msk)

        nzc = rows_pt // ch
        zrem = rows_pt - nzc * ch

        def zacc(r, _):
            pltpu.sync_copy(ab.at[0],
                            acc.at[pl.ds(sid * rows_pt + r * ch, ch)])
            return 0
        lax.fori_loop(0, nzc, zacc, 0, unroll=False)
        if zrem:
            pltpu.sync_copy(
                ab.at[0, pl.ds(0, zrem)],
                acc.at[pl.ds(sid * rows_pt + nzc * ch, zrem)])
        plsc.subcore_barrier()

        base0 = wid * epw

        def issue_chunk(t, slot):
            base = base0 + t * ch
            pltpu.sync_copy(ei_hbm.at[:, pl.ds(base, ch)], idx.at[slot])
            pltpu.async_copy(k_hbm.at[idx.at[slot, 0]], kg.at[slot],
                             semk[slot])
            pltpu.async_copy(q_hbm.at[idx.at[slot, 1]], qg.at[slot],
                             semq[slot])
            pltpu.async_copy(v_hbm.at[idx.at[slot, 0]], vg.at[slot],
                             semv[slot])
            pltpu.async_copy(p_hbm.at[pl.ds(base, ch)], pg.at[slot],
                             semp[slot])

        issue_chunk(0, 0)

        def unpack(w):
            return (plsc.bitcast(w << 16, jnp.float32),
                    plsc.bitcast(w & himask, jnp.float32))

        def body(pair, b):
            t = pair * 2 + b
            bn = b ^ 1
            base = base0 + t * ch

            def wait_prev_s():
                pltpu.make_async_copy(
                    ab.at[bn], acc.at[idx.at[bn, 1]], sems[bn]).wait()
            if b == 1:
                wait_prev_s()
            else:
                pl.when(pair >= 1)(wait_prev_s)

            def prefetch():
                issue_chunk(t + 1, bn)
            if b == 0:
                prefetch()
            else:
                pl.when(pair < npair - 1)(prefetch)

            pltpu.make_async_copy(k_hbm.at[idx.at[b, 0]], kg.at[b],
                                  semk[b]).wait()
            pltpu.make_async_copy(q_hbm.at[idx.at[b, 1]], qg.at[b],
                                  semq[b]).wait()
            pltpu.make_async_copy(v_hbm.at[idx.at[b, 0]], vg.at[b],
                                  semv[b]).wait()
            pltpu.make_async_copy(p_hbm.at[pl.ds(base, ch)], pg.at[b],
                                  semp[b]).wait()

            def wait_prev_e():
                pltpu.make_async_copy(
                    pb, e_hbm.at[pl.ds(base - ch, ch)], seme).wait()
            if b == 1:
                wait_prev_e()
            else:
                pl.when(pair >= 1)(wait_prev_e)

            @plsc.parallel_loop(0, ch)
            def p1(e):
                row = jnp.full((L,), e, jnp.int32)
                for j in range(H // 2):
                    sl = pl.ds(j * L, L)
                    klo, khi = unpack(kg[b, e, sl])
                    qlo, qhi = unpack(qg[b, e, sl])
                    plo, phi = unpack(pg[b, e, sl])
                    vlo, vhi = unpack(vg[b, e, sl])
                    sc_lo = klo * qlo * plo
                    sc_hi = khi * qhi * phi
                    pb[e, sl] = sc_lo
                    pb[e, pl.ds((j + 4) * L, L)] = sc_hi
                    r_lo = jnp.sum(sc_lo)
                    r_hi = jnp.sum(sc_hi)
                    s_lo = jnp.exp(jnp.full(
                        (L,), jnp.minimum(jnp.maximum(r_lo, -5.0), 5.0),
                        jnp.float32))
                    s_hi = jnp.exp(jnp.full(
                        (L,), jnp.minimum(jnp.maximum(r_hi, -5.0), 5.0),
                        jnp.float32))
                    ab[b, e, sl] = vlo * s_lo
                    ab[b, e, pl.ds((j + 4) * L, L)] = vhi * s_hi
                    plsc.store_scatter(
                        ab.at[b], [row, jnp.full((L,), d + j, jnp.int32)],
                        s_lo, mask=lane0)
                    plsc.store_scatter(
                        ab.at[b],
                        [row, jnp.full((L,), d + j + 4, jnp.int32)],
                        s_hi, mask=lane0)

            pltpu.async_copy(pb, e_hbm.at[pl.ds(base, ch)], seme)
            pltpu.async_copy(ab.at[b], acc.at[idx.at[b, 1]], sems[b],
                             add=True)

        def pair_loop(pair, _):
            body(pair, 0)
            body(pair, 1)
            return 0
        lax.fori_loop(0, npair, pair_loop, 0, unroll=False)

        pltpu.make_async_copy(
            pb, e_hbm.at[pl.ds(base0 + (nchunk - 1) * ch, ch)], seme).wait()
        pltpu.make_async_copy(ab.at[1], acc.at[idx.at[1, 1]], sems[1]).wait()

        plsc.subcore_barrier()

        def outcp(r, _):
            sl = pl.ds(sid * rows_pt + r * ch, ch)
            pltpu.sync_copy(acc.at[sl], ab.at[0])
            pltpu.sync_copy(ab.at[0], acc_hbm.at[cid, sl])
            return 0
        lax.fori_loop(0, nzc, outcp, 0, unroll=False)
        if zrem:
            sl = pl.ds(sid * rows_pt + nzc * ch, zrem)
            pltpu.sync_copy(acc.at[sl], ab.at[0, pl.ds(0, zrem)])
            pltpu.sync_copy(ab.at[0, pl.ds(0, zrem)], acc_hbm.at[cid, sl])

    return sc_kernel(q, k, v, p, ei)


def _final_body(acc_ref, s_ref, h_ref):
    a = acc_ref[0] + acc_ref[1]
    wv = a[:, 0:128]
    z8 = a[:, 128:136]
    zfull = jnp.dot(z8, s_ref[...], preferred_element_type=jnp.float32)
    h_ref[...] = wv / (zfull + 1e-6)


def _final_call(acc, bn):
    _, n, _ = acc.shape
    sel = np.zeros((H, 128), np.float32)
    for h in range(H):
        sel[h, h * DH:(h + 1) * DH] = 1.0
    sel = jnp.asarray(sel)
    return pl.pallas_call(
        _final_body,
        grid=(n // bn,),
        in_specs=[pl.BlockSpec((NC, bn, ACCW), lambda i: (0, i, 0)),
                  pl.BlockSpec((H, 128), lambda i: (0, 0))],
        out_specs=pl.BlockSpec((bn, 128), lambda i: (i, 0)),
        out_shape=jax.ShapeDtypeStruct((n, 128), jnp.float32),
    )(acc, sel)


def kernel(x, edge_attr, edge_index, WQ, WK, WV, WE):
    n, d = x.shape

    q, k, v = _qkv_call(x, WQ, WK, WV, bn=2000)
    p = _proj_call(edge_attr, WE, be=8000)
    e_out, acc = _sc_edge_call(q, k, v, p, edge_index, n, d, ch=40)
    h = _final_call(acc, bn=1000)
    return (h, e_out)
